# baseline (speedup 1.0000x reference)
import jax
import jax.numpy as jnp
from jax.experimental import pallas as pl
from jax.experimental.pallas import tpu as pltpu
from jax.experimental.pallas import tpu_sc as plsc

NC = 2
NS = 16
NW = NC * NS
LANES = 16
CB = 16
NROWS = (LANES + 1) * CB


def _scores_kernel(usr_ref, relt_ref, out_ref):
    out_ref[...] = jnp.dot(usr_ref[...], relt_ref[...],
                           preferred_element_type=jnp.float32)


def _user_rel_scores(usr_table, rel_t):
    nu = usr_table.shape[0]
    nr = rel_t.shape[1]
    return pl.pallas_call(
        _scores_kernel,
        out_shape=jax.ShapeDtypeStruct((nu, nr), jnp.float32),
    )(usr_table, rel_t)


def _sc_body(u_hbm, v_hbm, usr_hbm, ent_hbm, sp_hbm, ep_hbm, rp_hbm,
             uout_hbm, hout_hbm,
             u_all, v_all, urows, spk, mpk, rpk, nia, nib, nic, wmat,
             nrows, hrows, sem_a, sem_n, sem_uo, sem_ho):
    nb = u_hbm.shape[0]
    wid = jax.lax.axis_index("s") * NC + jax.lax.axis_index("c")
    per_w = nb // NW
    nchunks = per_w // CB
    wbase = wid * per_w
    iota = jax.lax.iota(jnp.int32, LANES)

    pltpu.sync_copy(u_hbm.at[pl.ds(wbase, per_w)], u_all)
    pltpu.sync_copy(v_hbm.at[pl.ds(wbase, per_w)], v_all)

    def fire_a(g, s):
        u_vec = u_all[pl.ds(g * CB, CB)]
        v_vec = v_all[pl.ds(g * CB, CB)]
        return [
            pltpu.async_copy(usr_hbm.at[u_vec], urows[s], sem_a[s]),
            pltpu.async_copy(sp_hbm.at[u_vec >> 3], spk[s], sem_a[s]),
            pltpu.async_copy(ep_hbm.at[v_vec >> 3], mpk[s], sem_a[s]),
            pltpu.async_copy(rp_hbm.at[v_vec >> 3], rpk[s], sem_a[s]),
        ]

    fire_a(0, 0)

    def unpack_fire_n(g, s):
        u_vec = u_all[pl.ds(g * CB, CB)]
        v_vec = v_all[pl.ds(g * CB, CB)]
        off_m = (v_vec & 7) << 4
        off_s = (u_vec & 7) << 4
        sc_n = []
        for n in range(LANES):
            ent_n = plsc.load_gather(mpk[s], [iota, off_m + n])
            rel_n = plsc.load_gather(rpk[s], [iota, off_m + n])
            if n < 8:
                nia[s][pl.ds(n * CB, CB)] = ent_n
            else:
                nib[s][pl.ds((n - 8) * CB, CB)] = ent_n
            sc_n.append(plsc.load_gather(spk[s], [iota, off_s + rel_n]))
        nic[s][...] = v_vec
        m = sc_n[0]
        for n in range(1, LANES):
            m = jnp.maximum(m, sc_n[n])
        e_n = [jnp.exp(x - m) for x in sc_n]
        tot = e_n[0]
        for n in range(1, LANES):
            tot = tot + e_n[n]
        inv = 1.0 / tot
        for n in range(LANES):
            wmat[s][pl.ds(n * CB, CB)] = e_n[n] * inv
        return [
            pltpu.async_copy(ent_hbm.at[nia[s]],
                             nrows[s].at[pl.ds(0, 128)], sem_n[s]),
            pltpu.async_copy(ent_hbm.at[nib[s]],
                             nrows[s].at[pl.ds(128, 128)], sem_n[s]),
            pltpu.async_copy(ent_hbm.at[nic[s]],
                             nrows[s].at[pl.ds(256, CB)], sem_n[s]),
        ]

    def n_descs(g, s):
        return [
            pltpu.make_async_copy(ent_hbm.at[nia[s]],
                                  nrows[s].at[pl.ds(0, 128)], sem_n[s]),
            pltpu.make_async_copy(ent_hbm.at[nib[s]],
                                  nrows[s].at[pl.ds(128, 128)], sem_n[s]),
            pltpu.make_async_copy(ent_hbm.at[nic[s]],
                                  nrows[s].at[pl.ds(256, CB)], sem_n[s]),
        ]

    def a_descs(g, s):
        u_vec = u_all[pl.ds(g * CB, CB)]
        v_vec = v_all[pl.ds(g * CB, CB)]
        return [
            pltpu.make_async_copy(usr_hbm.at[u_vec], urows[s], sem_a[s]),
            pltpu.make_async_copy(sp_hbm.at[u_vec >> 3], spk[s], sem_a[s]),
            pltpu.make_async_copy(ep_hbm.at[v_vec >> 3], mpk[s], sem_a[s]),
            pltpu.make_async_copy(rp_hbm.at[v_vec >> 3], rpk[s], sem_a[s]),
        ]

    def fire_uout(g, s):
        pltpu.async_copy(urows[s], uout_hbm.at[pl.ds(wbase + g * CB, CB)],
                         sem_uo[s])

    def drain_uout(g, s):
        pltpu.make_async_copy(urows[s],
                              uout_hbm.at[pl.ds(wbase + g * CB, CB)],
                              sem_uo[s]).wait()

    def drain_hout(g, s):
        pltpu.make_async_copy(hrows[s],
                              hout_hbm.at[pl.ds(wbase + g * CB, CB)],
                              sem_ho[s]).wait()

    def compute(g, s):
        @pl.loop(0, CB)
        def _item(i):
            wv = plsc.load_gather(wmat[s],
                                  [(iota << 4) + i])
            ws = [wv[n] for n in range(LANES)]
            for j in range(8):
                acc = nrows[s][256 + i, pl.ds(16 * j, 16)]
                for n in range(LANES):
                    acc = acc + ws[n] * nrows[s][n * CB + i, pl.ds(16 * j, 16)]
                hrows[s][i, pl.ds(16 * j, 16)] = acc

        pltpu.async_copy(hrows[s], hout_hbm.at[pl.ds(wbase + g * CB, CB)],
                         sem_ho[s])

    @pl.loop(0, nchunks, step=2)
    def _pair(g0):
        for ph in range(2):
            g = g0 + ph
            s = ph
            p = 1 - ph
            for d in a_descs(g, s):
                d.wait()
            unpack_fire_n(g, s)
            fire_uout(g, s)

            @pl.when(g + 1 < nchunks)
            def _():
                @pl.when(g >= 1)
                def _():
                    drain_uout(g - 1, p)
                fire_a(g + 1, p)

            @pl.when(g >= 1)
            def _():
                for d in n_descs(g - 1, p):
                    d.wait()

                @pl.when(g >= 3)
                def _():
                    drain_hout(g - 3, p)
                compute(g - 1, p)

    gl = nchunks - 1
    sl = gl & 1
    for d in n_descs(gl, sl):
        d.wait()
    drain_hout(gl - 2, sl)
    compute(gl, sl)
    drain_uout(gl - 1, 1 - sl)
    drain_uout(gl, sl)
    drain_hout(gl - 1, 1 - sl)
    drain_hout(gl, sl)


def _sc_gather_agg(u, v, usr_table, ent_table, s_pack, e_pack, r_pack):
    nb = u.shape[0]
    dim = usr_table.shape[1]
    per_w = nb // NW
    mesh = plsc.VectorSubcoreMesh(core_axis_name="c", subcore_axis_name="s",
                                  num_cores=NC, num_subcores=NS)
    f = pl.kernel(
        _sc_body,
        out_type=(jax.ShapeDtypeStruct((nb, dim), jnp.float32),
                  jax.ShapeDtypeStruct((nb, dim), jnp.float32)),
        mesh=mesh,
        compiler_params=pltpu.CompilerParams(needs_layout_passes=False),
        scratch_types=[
            pltpu.VMEM((per_w,), jnp.int32),
            pltpu.VMEM((per_w,), jnp.int32),
            [pltpu.VMEM((CB, dim), jnp.float32)] * 2,
            [pltpu.VMEM((CB, 128), jnp.float32)] * 2,
            [pltpu.VMEM((CB, 128), jnp.int32)] * 2,
            [pltpu.VMEM((CB, 128), jnp.int32)] * 2,
            [pltpu.VMEM((128,), jnp.int32)] * 2,
            [pltpu.VMEM((128,), jnp.int32)] * 2,
            [pltpu.VMEM((CB,), jnp.int32)] * 2,
            [pltpu.VMEM((LANES * CB,), jnp.float32)] * 2,
            [pltpu.VMEM((NROWS, dim), jnp.float32)] * 2,
            [pltpu.VMEM((CB, dim), jnp.float32)] * 2,
            [pltpu.SemaphoreType.DMA] * 2,
            [pltpu.SemaphoreType.DMA] * 2,
            [pltpu.SemaphoreType.DMA] * 2,
            [pltpu.SemaphoreType.DMA] * 2,
        ],
    )
    return f(u, v, usr_table, ent_table, s_pack, e_pack, r_pack)


def _final_kernel(user_ref, h_ref, wt_ref, out_ref):
    item = jnp.tanh(jnp.dot(h_ref[...], wt_ref[...],
                            preferred_element_type=jnp.float32))
    logits = jnp.sum(user_ref[...] * item, axis=1)
    out_ref[...] = jax.nn.sigmoid(logits)


def _final(user_emb, h, w_t):
    nb, dim = user_emb.shape
    blk = 2048
    grid = nb // blk
    out = pl.pallas_call(
        _final_kernel,
        grid=(grid,),
        in_specs=[
            pl.BlockSpec((blk, dim), lambda i: (i, 0)),
            pl.BlockSpec((blk, dim), lambda i: (i, 0)),
            pl.BlockSpec((dim, dim), lambda i: (0, 0)),
        ],
        out_specs=pl.BlockSpec((blk,), lambda i: (i,)),
        out_shape=jax.ShapeDtypeStruct((nb,), jnp.float32),
    )(user_emb, h, w_t)
    return out


def kernel(u, v, usr_table, ent_table, rel_table, W, adj_ent, adj_rel):
    s_all = _user_rel_scores(usr_table, rel_table.T)
    s_pack = s_all.reshape(-1, 128)
    e_pack = adj_ent.reshape(-1, 128)
    r_pack = adj_rel.reshape(-1, 128)
    user_emb, h = _sc_gather_agg(u, v, usr_table, ent_table, s_pack,
                                 e_pack, r_pack)
    return _final(user_emb, h, W.T)


# speedup vs baseline: 1.2034x; 1.2034x over previous
import jax
import jax.numpy as jnp
from jax.experimental import pallas as pl
from jax.experimental.pallas import tpu as pltpu
from jax.experimental.pallas import tpu_sc as plsc

NC = 2
NS = 16
NW = NC * NS
LANES = 16
CB = 16
NROWS = (LANES + 1) * CB


def _scores_kernel(usr_ref, relt_ref, out_ref):
    out_ref[...] = jnp.dot(usr_ref[...], relt_ref[...],
                           preferred_element_type=jnp.float32)


def _user_rel_scores(usr_table, rel_t):
    nu = usr_table.shape[0]
    nr = rel_t.shape[1]
    return pl.pallas_call(
        _scores_kernel,
        out_shape=jax.ShapeDtypeStruct((nu, nr), jnp.float32),
    )(usr_table, rel_t)


def _sc_body(u_hbm, v_hbm, usr_hbm, ent_hbm, sp_hbm, mp_hbm,
             uout_hbm, hout_hbm,
             u_all, v_all, urows, spk, mpk, nia, nib, nic, wmat,
             nrows, hrows, sem_a, sem_n, sem_uo, sem_ho):
    nb = u_hbm.shape[0]
    wid = jax.lax.axis_index("s") * NC + jax.lax.axis_index("c")
    per_w = nb // NW
    nchunks = per_w // CB
    wbase = wid * per_w
    iota = jax.lax.iota(jnp.int32, LANES)

    pltpu.sync_copy(u_hbm.at[pl.ds(wbase, per_w)], u_all)
    pltpu.sync_copy(v_hbm.at[pl.ds(wbase, per_w)], v_all)

    def fire_a(g, s):
        u_vec = u_all[pl.ds(g * CB, CB)]
        v_vec = v_all[pl.ds(g * CB, CB)]
        return [
            pltpu.async_copy(usr_hbm.at[u_vec], urows[s], sem_a[s]),
            pltpu.async_copy(sp_hbm.at[u_vec >> 3], spk[s], sem_a[s]),
            pltpu.async_copy(mp_hbm.at[v_vec >> 3], mpk[s], sem_a[s]),
        ]

    fire_a(0, 0)

    def unpack_fire_n(g, s):
        u_vec = u_all[pl.ds(g * CB, CB)]
        v_vec = v_all[pl.ds(g * CB, CB)]
        off_m = (v_vec & 7) << 4
        off_s = (u_vec & 7) << 4
        sc_n = []
        for n in range(LANES):
            c_n = plsc.load_gather(mpk[s], [iota, off_m + n])
            ent_n = c_n >> 4
            rel_n = c_n & 15
            if n < 8:
                nia[s][pl.ds(n * CB, CB)] = ent_n
            else:
                nib[s][pl.ds((n - 8) * CB, CB)] = ent_n
            sc_n.append(plsc.load_gather(spk[s], [iota, off_s + rel_n]))
        nic[s][...] = v_vec
        m = sc_n[0]
        for n in range(1, LANES):
            m = jnp.maximum(m, sc_n[n])
        e_n = [jnp.exp(x - m) for x in sc_n]
        tot = e_n[0]
        for n in range(1, LANES):
            tot = tot + e_n[n]
        inv = 1.0 / tot
        for n in range(LANES):
            wmat[s][pl.ds(n * CB, CB)] = e_n[n] * inv
        return [
            pltpu.async_copy(ent_hbm.at[nia[s]],
                             nrows[s].at[pl.ds(0, 128)], sem_n[s]),
            pltpu.async_copy(ent_hbm.at[nib[s]],
                             nrows[s].at[pl.ds(128, 128)], sem_n[s]),
            pltpu.async_copy(ent_hbm.at[nic[s]],
                             nrows[s].at[pl.ds(256, CB)], sem_n[s]),
        ]

    def n_descs(g, s):
        return [
            pltpu.make_async_copy(ent_hbm.at[nia[s]],
                                  nrows[s].at[pl.ds(0, 128)], sem_n[s]),
            pltpu.make_async_copy(ent_hbm.at[nib[s]],
                                  nrows[s].at[pl.ds(128, 128)], sem_n[s]),
            pltpu.make_async_copy(ent_hbm.at[nic[s]],
                                  nrows[s].at[pl.ds(256, CB)], sem_n[s]),
        ]

    def a_descs(g, s):
        u_vec = u_all[pl.ds(g * CB, CB)]
        v_vec = v_all[pl.ds(g * CB, CB)]
        return [
            pltpu.make_async_copy(usr_hbm.at[u_vec], urows[s], sem_a[s]),
            pltpu.make_async_copy(sp_hbm.at[u_vec >> 3], spk[s], sem_a[s]),
            pltpu.make_async_copy(mp_hbm.at[v_vec >> 3], mpk[s], sem_a[s]),
        ]

    def fire_uout(g, s):
        pltpu.async_copy(urows[s], uout_hbm.at[pl.ds(wbase + g * CB, CB)],
                         sem_uo[s])

    def drain_uout(g, s):
        pltpu.make_async_copy(urows[s],
                              uout_hbm.at[pl.ds(wbase + g * CB, CB)],
                              sem_uo[s]).wait()

    def drain_hout(g, s):
        pltpu.make_async_copy(hrows[s],
                              hout_hbm.at[pl.ds(wbase + g * CB, CB)],
                              sem_ho[s]).wait()

    def compute(g, s):
        @pl.loop(0, CB)
        def _item(i):
            wv = plsc.load_gather(wmat[s],
                                  [(iota << 4) + i])
            ws = [wv[n] for n in range(LANES)]
            for j in range(8):
                acc = nrows[s][256 + i, pl.ds(16 * j, 16)]
                for n in range(LANES):
                    acc = acc + ws[n] * nrows[s][n * CB + i, pl.ds(16 * j, 16)]
                hrows[s][i, pl.ds(16 * j, 16)] = acc

        pltpu.async_copy(hrows[s], hout_hbm.at[pl.ds(wbase + g * CB, CB)],
                         sem_ho[s])

    @pl.loop(0, nchunks, step=2)
    def _pair(g0):
        for ph in range(2):
            g = g0 + ph
            s = ph
            p = 1 - ph
            for d in a_descs(g, s):
                d.wait()
            unpack_fire_n(g, s)
            fire_uout(g, s)

            @pl.when(g + 1 < nchunks)
            def _():
                @pl.when(g >= 1)
                def _():
                    drain_uout(g - 1, p)
                fire_a(g + 1, p)

            @pl.when(g >= 1)
            def _():
                for d in n_descs(g - 1, p):
                    d.wait()

                @pl.when(g >= 3)
                def _():
                    drain_hout(g - 3, p)
                compute(g - 1, p)

    gl = nchunks - 1
    sl = gl & 1
    for d in n_descs(gl, sl):
        d.wait()
    drain_hout(gl - 2, sl)
    compute(gl, sl)
    drain_uout(gl - 1, 1 - sl)
    drain_uout(gl, sl)
    drain_hout(gl - 1, 1 - sl)
    drain_hout(gl, sl)


def _sc_gather_agg(u, v, usr_table, ent_table, s_pack, meta_pack):
    nb = u.shape[0]
    dim = usr_table.shape[1]
    per_w = nb // NW
    mesh = plsc.VectorSubcoreMesh(core_axis_name="c", subcore_axis_name="s",
                                  num_cores=NC, num_subcores=NS)
    f = pl.kernel(
        _sc_body,
        out_type=(jax.ShapeDtypeStruct((nb, dim), jnp.float32),
                  jax.ShapeDtypeStruct((nb, dim), jnp.float32)),
        mesh=mesh,
        compiler_params=pltpu.CompilerParams(needs_layout_passes=False),
        scratch_types=[
            pltpu.VMEM((per_w,), jnp.int32),
            pltpu.VMEM((per_w,), jnp.int32),
            [pltpu.VMEM((CB, dim), jnp.float32)] * 2,
            [pltpu.VMEM((CB, 128), jnp.float32)] * 2,
            [pltpu.VMEM((CB, 128), jnp.int32)] * 2,
            [pltpu.VMEM((128,), jnp.int32)] * 2,
            [pltpu.VMEM((128,), jnp.int32)] * 2,
            [pltpu.VMEM((CB,), jnp.int32)] * 2,
            [pltpu.VMEM((LANES * CB,), jnp.float32)] * 2,
            [pltpu.VMEM((NROWS, dim), jnp.float32)] * 2,
            [pltpu.VMEM((CB, dim), jnp.float32)] * 2,
            [pltpu.SemaphoreType.DMA] * 2,
            [pltpu.SemaphoreType.DMA] * 2,
            [pltpu.SemaphoreType.DMA] * 2,
            [pltpu.SemaphoreType.DMA] * 2,
        ],
    )
    return f(u, v, usr_table, ent_table, s_pack, meta_pack)


def _final_kernel(user_ref, h_ref, wt_ref, out_ref):
    item = jnp.tanh(jnp.dot(h_ref[...], wt_ref[...],
                            preferred_element_type=jnp.float32))
    logits = jnp.sum(user_ref[...] * item, axis=1)
    out_ref[...] = jax.nn.sigmoid(logits)


def _final(user_emb, h, w_t):
    nb, dim = user_emb.shape
    blk = 2048
    grid = nb // blk
    out = pl.pallas_call(
        _final_kernel,
        grid=(grid,),
        in_specs=[
            pl.BlockSpec((blk, dim), lambda i: (i, 0)),
            pl.BlockSpec((blk, dim), lambda i: (i, 0)),
            pl.BlockSpec((dim, dim), lambda i: (0, 0)),
        ],
        out_specs=pl.BlockSpec((blk,), lambda i: (i,)),
        out_shape=jax.ShapeDtypeStruct((nb,), jnp.float32),
    )(user_emb, h, w_t)
    return out


def kernel(u, v, usr_table, ent_table, rel_table, W, adj_ent, adj_rel):
    s_all = _user_rel_scores(usr_table, rel_table.T)
    s_pack = s_all.reshape(-1, 128)
    meta_pack = (adj_ent * 16 + adj_rel).reshape(-1, 128)
    user_emb, h = _sc_gather_agg(u, v, usr_table, ent_table, s_pack,
                                 meta_pack)
    return _final(user_emb, h, W.T)


# speedup vs baseline: 1.2131x; 1.0080x over previous
import jax
import jax.numpy as jnp
from jax.experimental import pallas as pl
from jax.experimental.pallas import tpu as pltpu
from jax.experimental.pallas import tpu_sc as plsc

NC = 2
NS = 16
NW = NC * NS
LANES = 16
CB = 16
NROWS = (LANES + 1) * CB


def _scores_kernel(usr_ref, relt_ref, out_ref):
    out_ref[...] = jnp.dot(usr_ref[...], relt_ref[...],
                           preferred_element_type=jnp.float32)


def _user_rel_scores(usr_table, rel_t):
    nu = usr_table.shape[0]
    nr = rel_t.shape[1]
    return pl.pallas_call(
        _scores_kernel,
        out_shape=jax.ShapeDtypeStruct((nu, nr), jnp.float32),
    )(usr_table, rel_t)


def _sc_body(u_hbm, v_hbm, usr_hbm, ent_hbm, sp_hbm, mp_hbm,
             uout_hbm, hout_hbm,
             u_all, v_all, urows, spk, mpk, nia, nib, nic, wmat,
             nrows, hrows, sem_a, sem_n, sem_uo, sem_ho):
    nb = u_hbm.shape[0]
    wid = jax.lax.axis_index("s") * NC + jax.lax.axis_index("c")
    per_w = nb // NW
    nchunks = per_w // CB
    wbase = wid * per_w
    iota = jax.lax.iota(jnp.int32, LANES)

    pltpu.sync_copy(u_hbm.at[pl.ds(wbase, per_w)], u_all)
    pltpu.sync_copy(v_hbm.at[pl.ds(wbase, per_w)], v_all)

    def fire_a(g, s):
        u_vec = u_all[pl.ds(g * CB, CB)]
        v_vec = v_all[pl.ds(g * CB, CB)]
        return [
            pltpu.async_copy(usr_hbm.at[u_vec], urows[s], sem_a[s]),
            pltpu.async_copy(sp_hbm.at[u_vec >> 3], spk[s], sem_a[s]),
            pltpu.async_copy(mp_hbm.at[v_vec >> 3], mpk[s], sem_a[s]),
        ]

    fire_a(0, 0)

    def unpack_fire_n(g, s):
        u_vec = u_all[pl.ds(g * CB, CB)]
        v_vec = v_all[pl.ds(g * CB, CB)]
        off_m = (v_vec & 7) << 4
        off_s = (u_vec & 7) << 4
        sc_n = []
        for n in range(LANES):
            c_n = plsc.load_gather(mpk[s], [iota, off_m + n])
            ent_n = c_n >> 4
            rel_n = c_n & 15
            if n < 8:
                nia[s][pl.ds(n * CB, CB)] = ent_n
            else:
                nib[s][pl.ds((n - 8) * CB, CB)] = ent_n
            sc_n.append(plsc.load_gather(spk[s], [iota, off_s + rel_n]))
        nic[s][...] = v_vec
        m = sc_n[0]
        for n in range(1, LANES):
            m = jnp.maximum(m, sc_n[n])
        e_n = [jnp.exp(x - m) for x in sc_n]
        tot = e_n[0]
        for n in range(1, LANES):
            tot = tot + e_n[n]
        inv = 1.0 / tot
        for n in range(LANES):
            wmat[s][pl.ds(n * CB, CB)] = e_n[n] * inv
        return [
            pltpu.async_copy(ent_hbm.at[nia[s]],
                             nrows[s].at[pl.ds(0, 128)], sem_n[s]),
            pltpu.async_copy(ent_hbm.at[nib[s]],
                             nrows[s].at[pl.ds(128, 128)], sem_n[s]),
            pltpu.async_copy(ent_hbm.at[nic[s]],
                             nrows[s].at[pl.ds(256, CB)], sem_n[s]),
        ]

    def n_descs(g, s):
        return [
            pltpu.make_async_copy(ent_hbm.at[nia[s]],
                                  nrows[s].at[pl.ds(0, 128)], sem_n[s]),
            pltpu.make_async_copy(ent_hbm.at[nib[s]],
                                  nrows[s].at[pl.ds(128, 128)], sem_n[s]),
            pltpu.make_async_copy(ent_hbm.at[nic[s]],
                                  nrows[s].at[pl.ds(256, CB)], sem_n[s]),
        ]

    def a_descs(g, s):
        u_vec = u_all[pl.ds(g * CB, CB)]
        v_vec = v_all[pl.ds(g * CB, CB)]
        return [
            pltpu.make_async_copy(usr_hbm.at[u_vec], urows[s], sem_a[s]),
            pltpu.make_async_copy(sp_hbm.at[u_vec >> 3], spk[s], sem_a[s]),
            pltpu.make_async_copy(mp_hbm.at[v_vec >> 3], mpk[s], sem_a[s]),
        ]

    def fire_uout(g, s):
        pltpu.async_copy(urows[s], uout_hbm.at[pl.ds(wbase + g * CB, CB)],
                         sem_uo[s])

    def drain_uout(g, s):
        pltpu.make_async_copy(urows[s],
                              uout_hbm.at[pl.ds(wbase + g * CB, CB)],
                              sem_uo[s]).wait()

    def drain_hout(g, s):
        pltpu.make_async_copy(hrows[s],
                              hout_hbm.at[pl.ds(wbase + g * CB, CB)],
                              sem_ho[s]).wait()

    def compute(g, s):
        @pl.loop(0, CB)
        def _item(i):
            wv = plsc.load_gather(wmat[s],
                                  [(iota << 4) + i])
            ws = [wv[n] for n in range(LANES)]
            for j in range(8):
                acc = nrows[s][256 + i, pl.ds(16 * j, 16)]
                for n in range(LANES):
                    acc = acc + ws[n] * nrows[s][n * CB + i, pl.ds(16 * j, 16)]
                hrows[s][i, pl.ds(16 * j, 16)] = acc

        pltpu.async_copy(hrows[s], hout_hbm.at[pl.ds(wbase + g * CB, CB)],
                         sem_ho[s])

    @pl.loop(0, nchunks, step=2)
    def _pair(g0):
        for ph in range(2):
            g = g0 + ph
            s = ph
            p = 1 - ph
            for d in a_descs(g, s):
                d.wait()
            unpack_fire_n(g, s)
            fire_uout(g, s)

            @pl.when(g + 1 < nchunks)
            def _():
                @pl.when(g >= 1)
                def _():
                    drain_uout(g - 1, p)
                fire_a(g + 1, p)

            @pl.when(g >= 1)
            def _():
                for d in n_descs(g - 1, p):
                    d.wait()

                @pl.when(g >= 3)
                def _():
                    drain_hout(g - 3, p)
                compute(g - 1, p)

    gl = nchunks - 1
    sl = gl & 1
    for d in n_descs(gl, sl):
        d.wait()
    drain_hout(gl - 2, sl)
    compute(gl, sl)
    drain_uout(gl - 1, 1 - sl)
    drain_uout(gl, sl)
    drain_hout(gl - 1, 1 - sl)
    drain_hout(gl, sl)


def _sc_gather_agg(u, v, usr_table, ent_table, s_pack, meta_pack):
    nb = u.shape[0]
    dim = usr_table.shape[1]
    per_w = nb // NW
    mesh = plsc.VectorSubcoreMesh(core_axis_name="c", subcore_axis_name="s",
                                  num_cores=NC, num_subcores=NS)
    f = pl.kernel(
        _sc_body,
        out_type=(jax.ShapeDtypeStruct((nb, dim), jnp.float32),
                  jax.ShapeDtypeStruct((nb, dim), jnp.float32)),
        mesh=mesh,
        compiler_params=pltpu.CompilerParams(needs_layout_passes=False),
        scratch_types=[
            pltpu.VMEM((per_w,), jnp.int32),
            pltpu.VMEM((per_w,), jnp.int32),
            [pltpu.VMEM((CB, dim), jnp.float32)] * 2,
            [pltpu.VMEM((CB, 128), jnp.float32)] * 2,
            [pltpu.VMEM((CB, 128), jnp.int32)] * 2,
            [pltpu.VMEM((128,), jnp.int32)] * 2,
            [pltpu.VMEM((128,), jnp.int32)] * 2,
            [pltpu.VMEM((CB,), jnp.int32)] * 2,
            [pltpu.VMEM((LANES * CB,), jnp.float32)] * 2,
            [pltpu.VMEM((NROWS, dim), jnp.float32)] * 2,
            [pltpu.VMEM((CB, dim), jnp.float32)] * 2,
            [pltpu.SemaphoreType.DMA] * 2,
            [pltpu.SemaphoreType.DMA] * 2,
            [pltpu.SemaphoreType.DMA] * 2,
            [pltpu.SemaphoreType.DMA] * 2,
        ],
    )
    return f(u, v, usr_table, ent_table, s_pack, meta_pack)


def _final_kernel(user_ref, h_ref, wt_ref, out_ref):
    item = jnp.tanh(jnp.dot(h_ref[...], wt_ref[...],
                            preferred_element_type=jnp.float32))
    logits = jnp.sum(user_ref[...] * item, axis=1)
    out_ref[...] = jax.nn.sigmoid(logits)


def _final(user_emb, h, w_t):
    nb, dim = user_emb.shape
    blk = 4096
    grid = nb // blk
    out = pl.pallas_call(
        _final_kernel,
        grid=(grid,),
        in_specs=[
            pl.BlockSpec((blk, dim), lambda i: (i, 0)),
            pl.BlockSpec((blk, dim), lambda i: (i, 0)),
            pl.BlockSpec((dim, dim), lambda i: (0, 0)),
        ],
        out_specs=pl.BlockSpec((blk,), lambda i: (i,)),
        out_shape=jax.ShapeDtypeStruct((nb,), jnp.float32),
    )(user_emb, h, w_t)
    return out


def kernel(u, v, usr_table, ent_table, rel_table, W, adj_ent, adj_rel):
    s_all = _user_rel_scores(usr_table, rel_table.T)
    s_pack = s_all.reshape(-1, 128)
    meta_pack = (adj_ent * 16 + adj_rel).reshape(-1, 128)
    user_emb, h = _sc_gather_agg(u, v, usr_table, ent_table, s_pack,
                                 meta_pack)
    return _final(user_emb, h, W.T)
